# baseline (speedup 1.0000x reference)
import jax
import jax.numpy as jnp
from jax import lax
from jax.experimental import pallas as pl
from jax.experimental.pallas import tpu as pltpu
from jax.experimental.pallas import tpu_sc as plsc

N_NODES = 10000
IN_DIM = 128
OUT_DIM = 128

NC, NS = 2, 16
NW = NC * NS
NPAD = 10112
NHIST = 10240
SLICE = NPAD // NS
HSLICE = NHIST // NS
CH = 128
NCHUNK = 80
IB = 4
NBLK = NCHUNK // IB
E_PAD = NW * NCHUNK * CH
BN = 2528

_sc_mesh = plsc.VectorSubcoreMesh(
    core_axis_name="c", subcore_axis_name="s", num_cores=NC, num_subcores=NS
)


def _deg_body(dst_hbm, deg_out, dst_v, ones_v, zrow_v, deg_sh):
    c = lax.axis_index("c")
    s = lax.axis_index("s")
    wid = c * NS + s
    zeros16 = jnp.zeros((16,), jnp.float32)
    ones16 = jnp.ones((16,), jnp.float32)
    for i in range(HSLICE // 16):
        zrow_v[pl.ds(i * 16, 16)] = zeros16
    for i in range(CH // 16):
        ones_v[pl.ds(i * 16, 16)] = ones16
    pltpu.sync_copy(zrow_v, deg_sh.at[pl.ds(s * HSLICE, HSLICE)])
    pltpu.sync_copy(dst_hbm.at[wid], dst_v)
    plsc.subcore_barrier()

    def chunk(j, carry):
        pltpu.sync_copy(ones_v, deg_sh.at[dst_v.at[j]], add=True)
        return carry

    lax.fori_loop(0, NCHUNK, chunk, 0)
    plsc.subcore_barrier()
    pltpu.sync_copy(
        deg_sh.at[pl.ds(s * HSLICE, HSLICE)],
        deg_out.at[c, pl.ds(s * HSLICE, HSLICE)],
    )


_deg_kernel = pl.kernel(
    _deg_body,
    out_type=jax.ShapeDtypeStruct((NC, NHIST), jnp.float32),
    mesh=_sc_mesh,
    scratch_types=[
        pltpu.VMEM((NCHUNK, CH), jnp.int32),
        pltpu.VMEM((CH,), jnp.float32),
        pltpu.VMEM((HSLICE,), jnp.float32),
        pltpu.VMEM_SHARED((NHIST,), jnp.float32),
    ],
)


def _agg_body(src_hbm, dst_hbm, h2_hbm, acc_out,
              srcb_v, dst_v, rows_v, zrow_v, acc_sh,
              semi, sem0, sem1, ssem0, ssem1):
    c = lax.axis_index("c")
    s = lax.axis_index("s")
    wid = c * NS + s
    with jax.named_scope("agg_zero"):
        zeros16 = jnp.zeros((16,), jnp.float32)
        for r in range(16):
            for q in range(OUT_DIM // 16):
                zrow_v[r, pl.ds(q * 16, 16)] = zeros16
        for zi in range(SLICE // 16):
            pltpu.sync_copy(zrow_v, acc_sh.at[pl.ds(s * SLICE + zi * 16, 16)])
        pltpu.sync_copy(
            zrow_v.at[pl.ds(0, SLICE % 16)],
            acc_sh.at[pl.ds(s * SLICE + (SLICE // 16) * 16, SLICE % 16)],
        )
        pltpu.sync_copy(src_hbm.at[wid, pl.ds(0, IB)], srcb_v.at[0])
        pltpu.async_copy(src_hbm.at[wid, pl.ds(IB, IB)], srcb_v.at[1], semi)
        pltpu.sync_copy(dst_hbm.at[wid], dst_v)
        plsc.subcore_barrier()

    with jax.named_scope("agg_prime"):
        pltpu.async_copy(h2_hbm.at[srcb_v.at[0, 0]], rows_v.at[0], sem0)
    sems = (sem0, sem1)
    ssems = (ssem0, ssem1)

    def blk_step(blk, carry):
        bp = lax.rem(blk, 2)
        nbp = lax.rem(blk + 1, 2)
        base = blk * IB

        @pl.when(blk < NBLK - 1)
        def _():
            pltpu.make_async_copy(
                src_hbm.at[wid, pl.ds(0, IB)], srcb_v.at[0], semi
            ).wait()

        for k in range(IB):
            b = k % 2
            with jax.named_scope("g_wait"):
                pltpu.make_async_copy(
                    h2_hbm.at[srcb_v.at[0, 0]], rows_v.at[b], sems[b]
                ).wait()
            pltpu.async_copy(
                rows_v.at[b], acc_sh.at[dst_v.at[base + k]], ssems[b], add=True
            )
            with jax.named_scope("s_wait"):
                if k == 0:
                    @pl.when(blk > 0)
                    def _():
                        pltpu.make_async_copy(
                            rows_v.at[1], acc_sh.at[dst_v.at[0]], ssems[1]
                        ).wait()
                else:
                    pltpu.make_async_copy(
                        rows_v.at[1 - b], acc_sh.at[dst_v.at[0]], ssems[1 - b]
                    ).wait()
            if k < IB - 1:
                pltpu.async_copy(
                    h2_hbm.at[srcb_v.at[bp, k + 1]], rows_v.at[1 - b],
                    sems[1 - b],
                )
            else:
                @pl.when(blk < NBLK - 2)
                def _():
                    pltpu.async_copy(
                        src_hbm.at[wid, pl.ds((blk + 2) * IB, IB)],
                        srcb_v.at[bp],
                        semi,
                    )

                @pl.when(blk < NBLK - 1)
                def _():
                    pltpu.async_copy(
                        h2_hbm.at[srcb_v.at[nbp, 0]], rows_v.at[1 - b],
                        sems[1 - b],
                    )

        return carry

    with jax.named_scope("agg_edges"):
        lax.fori_loop(0, NBLK, blk_step, 0)
        pltpu.make_async_copy(
            rows_v.at[1], acc_sh.at[dst_v.at[0]], ssems[1]
        ).wait()
        plsc.subcore_barrier()
    with jax.named_scope("agg_copyout"):
        pltpu.sync_copy(
            acc_sh.at[pl.ds(s * SLICE, SLICE)],
            acc_out.at[c, pl.ds(s * SLICE, SLICE), :],
        )


_agg_kernel = pl.kernel(
    _agg_body,
    out_type=jax.ShapeDtypeStruct((NC, NPAD, OUT_DIM), jnp.float32),
    mesh=_sc_mesh,
    scratch_types=[
        pltpu.VMEM((2, IB, CH), jnp.int32),
        pltpu.VMEM((NCHUNK, CH), jnp.int32),
        pltpu.VMEM((2, CH, OUT_DIM), jnp.float32),
        pltpu.VMEM((16, OUT_DIM), jnp.float32),
        pltpu.VMEM_SHARED((NPAD, OUT_DIM), jnp.float32),
        pltpu.SemaphoreType.DMA,
        pltpu.SemaphoreType.DMA,
        pltpu.SemaphoreType.DMA,
        pltpu.SemaphoreType.DMA,
        pltpu.SemaphoreType.DMA,
    ],
)


def _mm_body(degT_ref, x_ref, w_ref, h2_ref, dinv_ref):
    deg = degT_ref[:, 0:1] + degT_ref[:, 1:2] + 1.0
    dinv = lax.rsqrt(deg)
    h = jnp.dot(x_ref[...], w_ref[...], preferred_element_type=jnp.float32)
    h2_ref[...] = h * dinv
    dinv_ref[...] = dinv


def _mm_call(degT, x_p, W):
    return pl.pallas_call(
        _mm_body,
        grid=(NPAD // BN,),
        in_specs=[
            pl.BlockSpec((BN, 2), lambda i: (i, 0)),
            pl.BlockSpec((BN, IN_DIM), lambda i: (i, 0)),
            pl.BlockSpec((IN_DIM, OUT_DIM), lambda i: (0, 0)),
        ],
        out_specs=[
            pl.BlockSpec((BN, OUT_DIM), lambda i: (i, 0)),
            pl.BlockSpec((BN, 1), lambda i: (i, 0)),
        ],
        out_shape=[
            jax.ShapeDtypeStruct((NPAD, OUT_DIM), jnp.float32),
            jax.ShapeDtypeStruct((NPAD, 1), jnp.float32),
        ],
    )(degT, x_p, W)


def _out_body(h2_ref, a0_ref, a1_ref, dinv_ref, b_ref, out_ref, gout_ref,
              m_acc, s_acc):
    i = pl.program_id(0)
    o = (h2_ref[...] + a0_ref[...] + a1_ref[...]) * dinv_ref[...] + b_ref[0:1, :]
    o = jnp.maximum(o, 0.0)
    out_ref[...] = o
    rid = i * BN + lax.broadcasted_iota(jnp.int32, (BN, OUT_DIM), 0)
    om = jnp.where(rid < N_NODES, o, 0.0)

    @pl.when(i == 0)
    def _():
        m_acc[...] = jnp.zeros((8, OUT_DIM), jnp.float32)
        s_acc[...] = jnp.zeros((8, OUT_DIM), jnp.float32)

    m_acc[0:1, :] = jnp.maximum(m_acc[0:1, :], jnp.max(om, axis=0, keepdims=True))
    s_acc[0:1, :] = s_acc[0:1, :] + jnp.sum(om, axis=0, keepdims=True)

    @pl.when(i == pl.num_programs(0) - 1)
    def _():
        gout_ref[:, 0:OUT_DIM] = jnp.broadcast_to(m_acc[0:1, :], (8, OUT_DIM))
        gout_ref[:, OUT_DIM:] = jnp.broadcast_to(
            s_acc[0:1, :] * (1.0 / N_NODES), (8, OUT_DIM)
        )


def _out_call(h2, a0, a1, dinv, b8):
    return pl.pallas_call(
        _out_body,
        grid=(NPAD // BN,),
        in_specs=[
            pl.BlockSpec((BN, OUT_DIM), lambda i: (i, 0)),
            pl.BlockSpec((BN, OUT_DIM), lambda i: (i, 0)),
            pl.BlockSpec((BN, OUT_DIM), lambda i: (i, 0)),
            pl.BlockSpec((BN, 1), lambda i: (i, 0)),
            pl.BlockSpec((8, OUT_DIM), lambda i: (0, 0)),
        ],
        out_specs=[
            pl.BlockSpec((BN, OUT_DIM), lambda i: (i, 0)),
            pl.BlockSpec((8, 2 * OUT_DIM), lambda i: (0, 0)),
        ],
        out_shape=[
            jax.ShapeDtypeStruct((NPAD, OUT_DIM), jnp.float32),
            jax.ShapeDtypeStruct((8, 2 * OUT_DIM), jnp.float32),
        ],
        scratch_shapes=[
            pltpu.VMEM((8, OUT_DIM), jnp.float32),
            pltpu.VMEM((8, OUT_DIM), jnp.float32),
        ],
    )(h2, a0, a1, dinv, b8)


def kernel(x, edge_index, W, b):
    src = edge_index[0]
    dst = edge_index[1]
    e = src.shape[0]
    pad = E_PAD - e
    pad_dst = N_NODES + (jnp.arange(pad, dtype=jnp.int32) % (NPAD - N_NODES))
    pad_src = jnp.arange(pad, dtype=jnp.int32) % N_NODES
    src_p = jnp.concatenate([src, pad_src]).reshape(NW, NCHUNK, CH)
    dst_p = jnp.concatenate([dst, pad_dst]).reshape(NW, NCHUNK, CH)
    x_p = jnp.pad(x, ((0, NPAD - N_NODES), (0, 0)))

    degp = _deg_kernel(dst_p)
    degT = degp[:, :NPAD].T
    h2, dinv = _mm_call(degT, x_p, W)
    acc = _agg_kernel(src_p, dst_p, h2)
    b8 = jnp.broadcast_to(b.reshape(1, OUT_DIM), (8, OUT_DIM))
    out, gout = _out_call(h2, acc[0], acc[1], dinv, b8)
    return out[:N_NODES], gout[0:1, :]


# speedup vs baseline: 1.2268x; 1.2268x over previous
import jax
import jax.numpy as jnp
from jax import lax
from jax.experimental import pallas as pl
from jax.experimental.pallas import tpu as pltpu
from jax.experimental.pallas import tpu_sc as plsc

N_NODES = 10000
IN_DIM = 128
OUT_DIM = 128

NC, NS = 2, 16
NW = NC * NS
NPAD = 10112
NHIST = 10240
SLICE = NPAD // NS
HSLICE = NHIST // NS
CH = 128
TCHUNK = 80
NCHUNKS = NW * TCHUNK
IB = 4
NBLK = TCHUNK // IB
BN = 2000

_sc_mesh = plsc.VectorSubcoreMesh(
    core_axis_name="c", subcore_axis_name="s", num_cores=NC, num_subcores=NS
)


def _deg_body(e2_hbm, deg_out, dst_v, ones_v, zrow_v, deg_sh):
    c = lax.axis_index("c")
    s = lax.axis_index("s")
    wid = c * NS + s
    cbase = wid * TCHUNK
    zeros16 = jnp.zeros((16,), jnp.float32)
    ones16 = jnp.ones((16,), jnp.float32)
    for i in range(HSLICE // 16):
        zrow_v[pl.ds(i * 16, 16)] = zeros16
    for i in range(CH // 16):
        ones_v[pl.ds(i * 16, 16)] = ones16
    pltpu.sync_copy(zrow_v, deg_sh.at[pl.ds(s * HSLICE, HSLICE)])
    pltpu.sync_copy(e2_hbm.at[1, pl.ds(cbase, TCHUNK)], dst_v)
    plsc.subcore_barrier()

    def chunk(j, carry):
        pltpu.sync_copy(ones_v, deg_sh.at[dst_v.at[j]], add=True)
        return carry

    lax.fori_loop(0, TCHUNK, chunk, 0)
    plsc.subcore_barrier()
    pltpu.sync_copy(
        deg_sh.at[pl.ds(s * HSLICE, HSLICE)],
        deg_out.at[c, pl.ds(s * HSLICE, HSLICE)],
    )


_deg_kernel = pl.kernel(
    _deg_body,
    out_type=jax.ShapeDtypeStruct((NC, NHIST), jnp.float32),
    mesh=_sc_mesh,
    scratch_types=[
        pltpu.VMEM((TCHUNK, CH), jnp.int32),
        pltpu.VMEM((CH,), jnp.float32),
        pltpu.VMEM((HSLICE,), jnp.float32),
        pltpu.VMEM_SHARED((NHIST,), jnp.float32),
    ],
)


def _agg_body(e2_hbm, h2_hbm, acc_out,
              srcb_v, dst_v, rows_v, zrow_v, acc_sh, semi, sem0, sem1):
    c = lax.axis_index("c")
    s = lax.axis_index("s")
    wid = c * NS + s
    cbase = wid * TCHUNK
    zeros16 = jnp.zeros((16,), jnp.float32)
    for r in range(16):
        for q in range(OUT_DIM // 16):
            zrow_v[r, pl.ds(q * 16, 16)] = zeros16
    for zi in range(SLICE // 16):
        pltpu.sync_copy(zrow_v, acc_sh.at[pl.ds(s * SLICE + zi * 16, 16)])
    pltpu.sync_copy(
        zrow_v.at[pl.ds(0, SLICE % 16)],
        acc_sh.at[pl.ds(s * SLICE + (SLICE // 16) * 16, SLICE % 16)],
    )
    pltpu.sync_copy(e2_hbm.at[0, pl.ds(cbase, IB)], srcb_v.at[0])
    pltpu.async_copy(e2_hbm.at[0, pl.ds(cbase + IB, IB)], srcb_v.at[1], semi)
    pltpu.sync_copy(e2_hbm.at[1, pl.ds(cbase, TCHUNK)], dst_v)
    plsc.subcore_barrier()

    pltpu.async_copy(h2_hbm.at[srcb_v.at[0, 0]], rows_v.at[0], sem0)
    pltpu.async_copy(h2_hbm.at[srcb_v.at[0, 1]], rows_v.at[1], sem1)
    sems = (sem0, sem1)

    def blk_step(blk, carry):
        bp = lax.rem(blk, 2)
        nbp = lax.rem(blk + 1, 2)
        base = blk * IB

        @pl.when(blk < NBLK - 1)
        def _():
            pltpu.make_async_copy(
                e2_hbm.at[0, pl.ds(0, IB)], srcb_v.at[0], semi
            ).wait()

        for k in range(IB):
            sem = sems[k % 2]
            pltpu.make_async_copy(
                h2_hbm.at[srcb_v.at[0, 0]], rows_v.at[k % 2], sem
            ).wait()
            pltpu.sync_copy(
                rows_v.at[k % 2], acc_sh.at[dst_v.at[base + k]], add=True
            )
            if k < 2:
                pltpu.async_copy(
                    h2_hbm.at[srcb_v.at[bp, k + 2]], rows_v.at[k % 2], sem
                )
            else:
                if k == 3:
                    @pl.when(blk < NBLK - 2)
                    def _():
                        pltpu.async_copy(
                            e2_hbm.at[0, pl.ds(cbase + (blk + 2) * IB, IB)],
                            srcb_v.at[bp],
                            semi,
                        )

                @pl.when(blk < NBLK - 1)
                def _():
                    pltpu.async_copy(
                        h2_hbm.at[srcb_v.at[nbp, k - 2]], rows_v.at[k % 2], sem
                    )

        return carry

    with jax.named_scope("agg_edges"):
        lax.fori_loop(0, NBLK, blk_step, 0)
        plsc.subcore_barrier()
    with jax.named_scope("agg_copyout"):
        pltpu.sync_copy(
            acc_sh.at[pl.ds(s * SLICE, SLICE)],
            acc_out.at[c, pl.ds(s * SLICE, SLICE), :],
        )


_agg_kernel = pl.kernel(
    _agg_body,
    out_type=jax.ShapeDtypeStruct((NC, NPAD, OUT_DIM), jnp.float32),
    mesh=_sc_mesh,
    scratch_types=[
        pltpu.VMEM((2, IB, CH), jnp.int32),
        pltpu.VMEM((TCHUNK, CH), jnp.int32),
        pltpu.VMEM((2, CH, OUT_DIM), jnp.float32),
        pltpu.VMEM((16, OUT_DIM), jnp.float32),
        pltpu.VMEM_SHARED((NPAD, OUT_DIM), jnp.float32),
        pltpu.SemaphoreType.DMA,
        pltpu.SemaphoreType.DMA,
        pltpu.SemaphoreType.DMA,
    ],
)


def _mm_body(degT_ref, x_ref, w_ref, h2_ref, dinv_ref):
    deg = degT_ref[:, 0:1] + degT_ref[:, 1:2] + 1.0
    dinv = lax.rsqrt(deg)
    h = jnp.dot(x_ref[...], w_ref[...], preferred_element_type=jnp.float32)
    h2_ref[...] = h * dinv
    dinv_ref[...] = dinv


def _mm_call(degT, x, W):
    return pl.pallas_call(
        _mm_body,
        grid=(N_NODES // BN,),
        in_specs=[
            pl.BlockSpec((BN, 2), lambda i: (i, 0)),
            pl.BlockSpec((BN, IN_DIM), lambda i: (i, 0)),
            pl.BlockSpec((IN_DIM, OUT_DIM), lambda i: (0, 0)),
        ],
        out_specs=[
            pl.BlockSpec((BN, OUT_DIM), lambda i: (i, 0)),
            pl.BlockSpec((BN, 1), lambda i: (i, 0)),
        ],
        out_shape=[
            jax.ShapeDtypeStruct((N_NODES, OUT_DIM), jnp.float32),
            jax.ShapeDtypeStruct((N_NODES, 1), jnp.float32),
        ],
    )(degT, x, W)


def _out_body(h2_ref, a0_ref, a1_ref, dinv_ref, b_ref, out_ref, gout_ref,
              m_acc, s_acc):
    i = pl.program_id(0)
    o = (h2_ref[...] + a0_ref[0] + a1_ref[0]) * dinv_ref[...] + b_ref[0:1, :]
    o = jnp.maximum(o, 0.0)
    out_ref[...] = o

    @pl.when(i == 0)
    def _():
        m_acc[...] = jnp.zeros((8, OUT_DIM), jnp.float32)
        s_acc[...] = jnp.zeros((8, OUT_DIM), jnp.float32)

    m_acc[0:1, :] = jnp.maximum(m_acc[0:1, :], jnp.max(o, axis=0, keepdims=True))
    s_acc[0:1, :] = s_acc[0:1, :] + jnp.sum(o, axis=0, keepdims=True)

    @pl.when(i == pl.num_programs(0) - 1)
    def _():
        gout_ref[:, 0:OUT_DIM] = jnp.broadcast_to(m_acc[0:1, :], (8, OUT_DIM))
        gout_ref[:, OUT_DIM:] = jnp.broadcast_to(
            s_acc[0:1, :] * (1.0 / N_NODES), (8, OUT_DIM)
        )


def _out_call(h2, acc, dinv, b8):
    return pl.pallas_call(
        _out_body,
        grid=(N_NODES // BN,),
        in_specs=[
            pl.BlockSpec((BN, OUT_DIM), lambda i: (i, 0)),
            pl.BlockSpec((1, BN, OUT_DIM), lambda i: (0, i, 0)),
            pl.BlockSpec((1, BN, OUT_DIM), lambda i: (1, i, 0)),
            pl.BlockSpec((BN, 1), lambda i: (i, 0)),
            pl.BlockSpec((8, OUT_DIM), lambda i: (0, 0)),
        ],
        out_specs=[
            pl.BlockSpec((BN, OUT_DIM), lambda i: (i, 0)),
            pl.BlockSpec((8, 2 * OUT_DIM), lambda i: (0, 0)),
        ],
        out_shape=[
            jax.ShapeDtypeStruct((N_NODES, OUT_DIM), jnp.float32),
            jax.ShapeDtypeStruct((8, 2 * OUT_DIM), jnp.float32),
        ],
        scratch_shapes=[
            pltpu.VMEM((8, OUT_DIM), jnp.float32),
            pltpu.VMEM((8, OUT_DIM), jnp.float32),
        ],
    )(h2, acc, acc, dinv, b8)


def kernel(x, edge_index, W, b):
    e = edge_index.shape[1]
    pad = NCHUNKS * CH - e
    pad_src = jnp.arange(pad, dtype=jnp.int32) % N_NODES
    pad_dst = N_NODES + (jnp.arange(pad, dtype=jnp.int32) % (NPAD - N_NODES))
    e2 = jnp.concatenate(
        [edge_index, jnp.stack([pad_src, pad_dst])], axis=1
    ).reshape(2, NCHUNKS, CH)

    degp = _deg_kernel(e2)
    degT = degp[:, :N_NODES].T
    h2, dinv = _mm_call(degT, x, W)
    acc = _agg_kernel(e2, h2)
    b8 = jnp.broadcast_to(b.reshape(1, OUT_DIM), (8, OUT_DIM))
    out, gout = _out_call(h2, acc, dinv, b8)
    return out, gout[0:1, :]
